# Initial kernel scaffold; baseline (speedup 1.0000x reference)
import jax
import jax.numpy as jnp
from jax.experimental import pallas as pl


def kernel(x, w_p, b_p):
    raise NotImplementedError("write your pallas kernel here")


# speedup vs baseline: 1.1716x; 1.1716x over previous
import jax
import jax.numpy as jnp
from jax.experimental import pallas as pl
from jax.experimental.pallas import tpu as pltpu


def _round_up(x: int, m: int) -> int:
    return ((x + m - 1) // m) * m


_TM = 512


def _fused_kernel(x_ref, w_ref, b_ref, o_ref):
    xb = x_ref[...].astype(jnp.bfloat16)
    l1 = (
        jnp.dot(xb, w_ref[...], preferred_element_type=jnp.float32)
        + b_ref[...]
    )
    o_ref[...] = l1 * ((jnp.clip(l1, 0.0, 6.0) + 3.0) * (1.0 / 6.0))


@jax.jit
def kernel(x, w_p, b_p):
    B, in_f = x.shape
    INp = w_p.shape[0]
    OUTp = w_p.shape[1]

    tm = min(_TM, _round_up(B, 8))
    Bp = _round_up(B, tm)
    if (Bp != B) or (INp != in_f):
        x = jnp.pad(x, ((0, Bp - B), (0, INp - in_f)))

    w_bf = w_p.astype(jnp.bfloat16)

    return pl.pallas_call(
        _fused_kernel,
        out_shape=jax.ShapeDtypeStruct((Bp, OUTp), jnp.float32),
        grid=(Bp // tm,),
        in_specs=[
            pl.BlockSpec((tm, INp), lambda i: (i, 0)),
            pl.BlockSpec((INp, OUTp), lambda i: (0, 0)),
            pl.BlockSpec((1, OUTp), lambda i: (0, 0)),
        ],
        out_specs=pl.BlockSpec((tm, OUTp), lambda i: (i, 0)),
        compiler_params=pltpu.CompilerParams(
            dimension_semantics=("parallel",),
        ),
    )(x, w_bf, b_p)


# speedup vs baseline: 1.3197x; 1.1265x over previous
import jax
import jax.numpy as jnp
from jax.experimental import pallas as pl
from jax.experimental.pallas import tpu as pltpu


def _round_up(x: int, m: int) -> int:
    return ((x + m - 1) // m) * m


_TM = 1024


def _fused_kernel(x_ref, w_ref, b_ref, o_ref):
    xb = x_ref[...].astype(jnp.bfloat16)
    l1 = (
        jnp.dot(xb, w_ref[...], preferred_element_type=jnp.float32)
        + b_ref[...]
    )
    o_ref[...] = l1 * ((jnp.clip(l1, 0.0, 6.0) + 3.0) * (1.0 / 6.0))


@jax.jit
def kernel(x, w_p, b_p):
    B, in_f = x.shape
    INp = w_p.shape[0]
    OUTp = w_p.shape[1]

    tm = min(_TM, _round_up(B, 8))
    Bp = _round_up(B, tm)
    if (Bp != B) or (INp != in_f):
        x = jnp.pad(x, ((0, Bp - B), (0, INp - in_f)))

    w_bf = w_p.astype(jnp.bfloat16)

    return pl.pallas_call(
        _fused_kernel,
        out_shape=jax.ShapeDtypeStruct((Bp, OUTp), jnp.float32),
        grid=(Bp // tm,),
        in_specs=[
            pl.BlockSpec((tm, INp), lambda i: (i, 0)),
            pl.BlockSpec((INp, OUTp), lambda i: (0, 0)),
            pl.BlockSpec((1, OUTp), lambda i: (0, 0)),
        ],
        out_specs=pl.BlockSpec((tm, OUTp), lambda i: (i, 0)),
        compiler_params=pltpu.CompilerParams(
            dimension_semantics=("parallel",),
        ),
    )(x, w_bf, b_p)
